# baseline (speedup 1.0000x reference)
import functools

import jax
import jax.numpy as jnp
from jax import lax
from jax.experimental import pallas as pl
from jax.experimental.pallas import tpu as pltpu
from jax.experimental.pallas import tpu_sc as plsc

N_ROWS = 16384
N_COLS = 256
NC = 2
NS = 16
L = 16
NW = NC * NS
ROWS_PER_W = N_ROWS // NW
G = 64
N_GROUPS = ROWS_PER_W // G


def _sc_excl_cumsum(x_flat):
    mesh = plsc.VectorSubcoreMesh(core_axis_name="c", subcore_axis_name="s")

    @functools.partial(
        pl.kernel,
        mesh=mesh,
        out_type=jax.ShapeDtypeStruct((N_ROWS * N_COLS,), jnp.float32),
        scratch_types=[
            pltpu.VMEM((G * N_COLS,), jnp.float32),
            pltpu.VMEM((G * N_COLS,), jnp.float32),
        ],
        compiler_params=pltpu.CompilerParams(needs_layout_passes=False),
    )
    def k(x_hbm, out_hbm, ibuf, obuf):
        wid = lax.axis_index("s") * NC + lax.axis_index("c")
        base = wid * (ROWS_PER_W * N_COLS)
        row_base = lax.iota(jnp.int32, L) * N_COLS

        def group(g, carry):
            goff = base + g * (G * N_COLS)
            pltpu.sync_copy(x_hbm.at[pl.ds(goff, G * N_COLS)], ibuf)
            for sg in range(G // L):
                sg_base = row_base + sg * (L * N_COLS)

                def col(c, acc):
                    idx = sg_base + c
                    v = plsc.load_gather(ibuf, [idx])
                    plsc.store_scatter(obuf, [idx], acc)
                    return acc + v

                lax.fori_loop(0, N_COLS, col, jnp.zeros((L,), jnp.float32))
            pltpu.sync_copy(obuf, out_hbm.at[pl.ds(goff, G * N_COLS)])
            return carry

        lax.fori_loop(0, N_GROUPS, group, 0)

    return k(x_flat)


def kernel(x):
    out_flat = _sc_excl_cumsum(x.reshape(-1))
    return out_flat.reshape(N_ROWS, N_COLS)
